# baseline (speedup 1.0000x reference)
import jax
import jax.numpy as jnp
from jax import lax
from jax.experimental import pallas as pl
from jax.experimental.pallas import tpu as pltpu
from jax.experimental.pallas import tpu_sc as plsc

DIM = 64
NC, NS = 2, 16
NW = NC * NS
CHUNK = 128
GB = 2


def _body(bpw, H, NB):
    ngroups = bpw // GB
    npairs = ngroups // 2

    def body(x_hbm, w_hbm, out_hbm, idx_v, raw_a, raw_b, t_a, t_b,
             gsem_a, gsem_b, wsem_a, wsem_b):
        wid = lax.axis_index("s") * NC + lax.axis_index("c")
        b0 = wid * bpw
        pltpu.sync_copy(x_hbm.at[pl.ds(b0, bpw)], idx_v)

        iota = lax.iota(jnp.int32, 16)

        def fire(raw, gsem, g):
            for j in range(GB):
                pltpu.async_copy(
                    w_hbm.at[idx_v.at[g * GB + j]],
                    raw.at[pl.ds(j * CHUNK, CHUNK)], gsem)

        def drain_gather(raw, gsem):
            pltpu.make_async_copy(
                w_hbm.at[pl.ds(0, GB * CHUNK)], raw, gsem).wait()

        def transpose(raw, t):
            def step(tt, carry):
                row_idx = iota + tt * 16
                j = tt // 8
                bg = tt % 8
                off = bg * 16
                for d in range(DIM):
                    vec = plsc.load_gather(
                        raw, [row_idx, jnp.full((16,), d, jnp.int32)])
                    t[d // 8, j, pl.ds((d % 8) * CHUNK + off, 16)] = vec
                return carry
            lax.fori_loop(0, 16, step, 0)

        def start_writes(t, wsem, g):
            blk = b0 + g * GB
            h = blk // NB
            tc = blk % NB
            for tr in range(8):
                pltpu.async_copy(
                    t.at[tr], out_hbm.at[pl.ds(h * 1024 + tr * NB + tc, GB)],
                    wsem)

        def wait_writes(t, wsem):
            for tr in range(8):
                pltpu.make_async_copy(
                    t.at[tr], out_hbm.at[pl.ds(0, GB)], wsem).wait()

        fire(raw_a, gsem_a, 0)

        def pair(p, carry):
            ga = 2 * p
            gb = 2 * p + 1
            drain_gather(raw_a, gsem_a)
            fire(raw_b, gsem_b, gb)

            @pl.when(p > 0)
            def _():
                wait_writes(t_a, wsem_a)

            transpose(raw_a, t_a)
            start_writes(t_a, wsem_a, ga)

            drain_gather(raw_b, gsem_b)

            @pl.when(p < npairs - 1)
            def _():
                fire(raw_a, gsem_a, ga + 2)

            @pl.when(p > 0)
            def _():
                wait_writes(t_b, wsem_b)

            transpose(raw_b, t_b)
            start_writes(t_b, wsem_b, gb)
            return carry

        lax.fori_loop(0, npairs, pair, 0)
        wait_writes(t_a, wsem_a)
        wait_writes(t_b, wsem_b)
    return body


def kernel(x, weight):
    B, H = x.shape
    V, D = weight.shape
    assert D == DIM and B % CHUNK == 0
    NB = B // CHUNK
    nblocks = H * NB
    assert nblocks % (NW * 2 * GB) == 0
    bpw = nblocks // NW
    xf = x.T.reshape(nblocks, CHUNK).astype(jnp.int32)
    mesh = plsc.VectorSubcoreMesh(
        core_axis_name="c", subcore_axis_name="s", num_cores=NC, num_subcores=NS
    )
    out = pl.kernel(
        _body(bpw, H, NB),
        out_type=jax.ShapeDtypeStruct((nblocks * 8, 1024), jnp.float32),
        mesh=mesh,
        compiler_params=pltpu.CompilerParams(
            use_tc_tiling_on_sc=False, needs_layout_passes=False),
        scratch_types=[
            pltpu.VMEM((bpw, CHUNK), jnp.int32),
            pltpu.VMEM((GB * CHUNK, DIM), jnp.float32),
            pltpu.VMEM((GB * CHUNK, DIM), jnp.float32),
            pltpu.VMEM((8, GB, 1024), jnp.float32),
            pltpu.VMEM((8, GB, 1024), jnp.float32),
            pltpu.SemaphoreType.DMA,
            pltpu.SemaphoreType.DMA,
            pltpu.SemaphoreType.DMA,
            pltpu.SemaphoreType.DMA,
        ],
    )(xf, weight)
    return (out.reshape(H, 8, NB, 8, CHUNK)
               .transpose(2, 4, 0, 1, 3)
               .reshape(B, H, DIM))


# speedup vs baseline: 1.3951x; 1.3951x over previous
import jax
import jax.numpy as jnp
from jax import lax
from jax.experimental import pallas as pl
from jax.experimental.pallas import tpu as pltpu
from jax.experimental.pallas import tpu_sc as plsc

DIM = 64
NC, NS = 2, 16
NW = NC * NS
CHUNK = 128
GB = 2


def _body(bpw, H, NB):
    ngroups = bpw // GB
    npairs = ngroups // 2

    def body(x_hbm, w_hbm, out_hbm, idx_v, raw_a, raw_b, t_a, t_b,
             gsem_a, gsem_b, wsem_a, wsem_b):
        wid = lax.axis_index("s") * NC + lax.axis_index("c")
        b0 = wid * bpw
        pltpu.sync_copy(x_hbm.at[pl.ds(b0, bpw)], idx_v)

        iota = lax.iota(jnp.int32, 16)

        def fire(raw, gsem, g):
            for j in range(GB):
                pltpu.async_copy(
                    w_hbm.at[idx_v.at[g * GB + j]],
                    raw.at[pl.ds(j * CHUNK, CHUNK)], gsem)

        def drain_gather(raw, gsem):
            pltpu.make_async_copy(
                w_hbm.at[pl.ds(0, GB * CHUNK)], raw, gsem).wait()

        def transpose(raw, t):
            BATCH = 16

            def step(tt, carry):
                row_idx = iota + tt * 16
                j = tt // 8
                off = (tt % 8) * 16
                prev = None
                for k in range(DIM // BATCH):
                    cur = [
                        (d, plsc.load_gather(
                            raw, [row_idx, jnp.full((16,), d, jnp.int32)]))
                        for d in range(k * BATCH, (k + 1) * BATCH)
                    ]
                    if prev is not None:
                        for d, vec in prev:
                            t[d // 8, j, pl.ds((d % 8) * CHUNK + off, 16)] = vec
                    prev = cur
                for d, vec in prev:
                    t[d // 8, j, pl.ds((d % 8) * CHUNK + off, 16)] = vec
                return carry
            lax.fori_loop(0, 16, step, 0)

        def start_writes(t, wsem, g):
            blk = b0 + g * GB
            h = blk // NB
            tc = blk % NB
            for tr in range(8):
                pltpu.async_copy(
                    t.at[tr], out_hbm.at[pl.ds(h * 1024 + tr * NB + tc, GB)],
                    wsem)

        def wait_writes(t, wsem):
            for tr in range(8):
                pltpu.make_async_copy(
                    t.at[tr], out_hbm.at[pl.ds(0, GB)], wsem).wait()

        fire(raw_a, gsem_a, 0)

        def pair(p, carry):
            ga = 2 * p
            gb = 2 * p + 1
            drain_gather(raw_a, gsem_a)
            fire(raw_b, gsem_b, gb)

            @pl.when(p > 0)
            def _():
                wait_writes(t_a, wsem_a)

            transpose(raw_a, t_a)
            start_writes(t_a, wsem_a, ga)

            drain_gather(raw_b, gsem_b)

            @pl.when(p < npairs - 1)
            def _():
                fire(raw_a, gsem_a, ga + 2)

            @pl.when(p > 0)
            def _():
                wait_writes(t_b, wsem_b)

            transpose(raw_b, t_b)
            start_writes(t_b, wsem_b, gb)
            return carry

        lax.fori_loop(0, npairs, pair, 0)
        wait_writes(t_a, wsem_a)
        wait_writes(t_b, wsem_b)
    return body


def kernel(x, weight):
    B, H = x.shape
    V, D = weight.shape
    assert D == DIM and B % CHUNK == 0
    NB = B // CHUNK
    nblocks = H * NB
    assert nblocks % (NW * 2 * GB) == 0
    bpw = nblocks // NW
    xf = x.T.reshape(nblocks, CHUNK).astype(jnp.int32)
    mesh = plsc.VectorSubcoreMesh(
        core_axis_name="c", subcore_axis_name="s", num_cores=NC, num_subcores=NS
    )
    out = pl.kernel(
        _body(bpw, H, NB),
        out_type=jax.ShapeDtypeStruct((nblocks * 8, 1024), jnp.float32),
        mesh=mesh,
        compiler_params=pltpu.CompilerParams(
            use_tc_tiling_on_sc=False, needs_layout_passes=False),
        scratch_types=[
            pltpu.VMEM((bpw, CHUNK), jnp.int32),
            pltpu.VMEM((GB * CHUNK, DIM), jnp.float32),
            pltpu.VMEM((GB * CHUNK, DIM), jnp.float32),
            pltpu.VMEM((8, GB, 1024), jnp.float32),
            pltpu.VMEM((8, GB, 1024), jnp.float32),
            pltpu.SemaphoreType.DMA,
            pltpu.SemaphoreType.DMA,
            pltpu.SemaphoreType.DMA,
            pltpu.SemaphoreType.DMA,
        ],
    )(xf, weight)
    return (out.reshape(H, 8, NB, 8, CHUNK)
               .transpose(2, 4, 0, 1, 3)
               .reshape(B, H, DIM))
